# Initial kernel scaffold; baseline (speedup 1.0000x reference)
import jax
import jax.numpy as jnp
from jax.experimental import pallas as pl


def kernel(x, table):
    raise NotImplementedError("write your pallas kernel here")


# speedup vs baseline: 3.6945x; 3.6945x over previous
import jax
import jax.numpy as jnp
from jax import lax
from jax.experimental import pallas as pl
from jax.experimental.pallas import tpu as pltpu, tpu_sc as plsc

EMBED_DIM = 768
NUM_CORES = 2
NUM_SUBCORES = 16
NUM_WORKERS = NUM_CORES * NUM_SUBCORES
CHUNK = 64


def _sc_body(rows_per_w, batch, table_hbm, out_hbm, buf):
    wid = lax.axis_index("s") * NUM_CORES + lax.axis_index("c")
    base = wid * rows_per_w
    for j in range(rows_per_w // CHUNK):
        r0 = base + j * CHUNK
        pltpu.sync_copy(table_hbm.at[pl.ds(r0, CHUNK)], buf)
        for b in range(batch):
            pltpu.sync_copy(buf, out_hbm.at[b, pl.ds(r0, CHUNK)])


def kernel(x, table):
    batch, seq = x.shape
    max_len, d = table.shape
    assert seq == max_len and d == EMBED_DIM
    rows_per_w = max_len // NUM_WORKERS

    mesh = plsc.VectorSubcoreMesh(core_axis_name="c", subcore_axis_name="s")
    run = pl.kernel(
        lambda *refs: _sc_body(rows_per_w, batch, *refs),
        out_type=jax.ShapeDtypeStruct((batch, seq, d), jnp.float32),
        mesh=mesh,
        scratch_types=[pltpu.VMEM((CHUNK, d), jnp.float32)],
    )
    return run(table)


# speedup vs baseline: 3.7798x; 1.0231x over previous
import jax
import jax.numpy as jnp
from jax import lax
from jax.experimental import pallas as pl
from jax.experimental.pallas import tpu as pltpu, tpu_sc as plsc

EMBED_DIM = 768
NUM_CORES = 2
NUM_SUBCORES = 16
NUM_WORKERS = NUM_CORES * NUM_SUBCORES
CHUNK = 64


def _sc_body(rows_per_w, batch, table_hbm, out_hbm, buf0, buf1, gsem0, gsem1,
             ssem0, ssem1):
    wid = lax.axis_index("s") * NUM_CORES + lax.axis_index("c")
    base = wid * rows_per_w
    n = rows_per_w // CHUNK
    bufs, gsems, ssems = [buf0, buf1], [gsem0, gsem1], [ssem0, ssem1]
    gathers = [None] * n
    scatters = [[] for _ in range(n)]

    def start_gather(j):
        gathers[j] = pltpu.async_copy(
            table_hbm.at[pl.ds(base + j * CHUNK, CHUNK)], bufs[j % 2],
            gsems[j % 2])

    start_gather(0)
    for j in range(n):
        if j + 1 < n:
            for c in scatters[j - 1] if j >= 1 else ():
                c.wait()
            start_gather(j + 1)
        gathers[j].wait()
        for b in range(batch):
            scatters[j].append(pltpu.async_copy(
                bufs[j % 2], out_hbm.at[b, pl.ds(base + j * CHUNK, CHUNK)],
                ssems[j % 2]))
    for c in scatters[n - 2] + scatters[n - 1]:
        c.wait()


def kernel(x, table):
    batch, seq = x.shape
    max_len, d = table.shape
    assert seq == max_len and d == EMBED_DIM
    rows_per_w = max_len // NUM_WORKERS

    mesh = plsc.VectorSubcoreMesh(core_axis_name="c", subcore_axis_name="s")
    run = pl.kernel(
        lambda *refs: _sc_body(rows_per_w, batch, *refs),
        out_type=jax.ShapeDtypeStruct((batch, seq, d), jnp.float32),
        mesh=mesh,
        scratch_types=[
            pltpu.VMEM((CHUNK, d), jnp.float32),
            pltpu.VMEM((CHUNK, d), jnp.float32),
            pltpu.SemaphoreType.DMA,
            pltpu.SemaphoreType.DMA,
            pltpu.SemaphoreType.DMA,
            pltpu.SemaphoreType.DMA,
        ],
    )
    return run(table)


# speedup vs baseline: 5.5800x; 1.4763x over previous
import jax
import jax.numpy as jnp
from jax import lax
from jax.experimental import pallas as pl
from jax.experimental.pallas import tpu as pltpu, tpu_sc as plsc

EMBED_DIM = 768
NUM_CORES = 2
NUM_SUBCORES = 16
NUM_WORKERS = NUM_CORES * NUM_SUBCORES
CHUNK = 64


def _sc_body(rows_per_w, batch, table_hbm, out_hbm, buf0, buf1, gsem0, gsem1,
             ssem0, ssem1):
    wid = lax.axis_index("s") * NUM_CORES + lax.axis_index("c")
    base = wid * rows_per_w
    n = rows_per_w // CHUNK
    bufs, gsems, ssems = [buf0, buf1], [gsem0, gsem1], [ssem0, ssem1]
    gathers = [None] * n
    scatters = [[] for _ in range(n)]

    def start_gather(j):
        gathers[j] = pltpu.async_copy(
            table_hbm.at[pl.ds(base + j * CHUNK, CHUNK)], bufs[j % 2],
            gsems[j % 2])

    start_gather(0)
    for j in range(n):
        if j + 1 < n:
            for c in scatters[j - 1] if j >= 1 else ():
                c.wait()
            start_gather(j + 1)
        gathers[j].wait()
        for b in range(batch):
            scatters[j].append(pltpu.async_copy(
                bufs[j % 2], out_hbm.at[b, pl.ds(base + j * CHUNK, CHUNK)],
                ssems[j % 2]))
    for c in scatters[n - 2] + scatters[n - 1]:
        c.wait()


def _tc_body(batch, table_ref, out_ref):
    out_ref[...] = jnp.broadcast_to(
        table_ref[...][None], (batch,) + table_ref.shape)


def kernel(x, table):
    batch, seq = x.shape
    max_len, d = table.shape
    assert seq == max_len and d == EMBED_DIM
    bs = 512
    return pl.pallas_call(
        lambda t, o: _tc_body(batch, t, o),
        grid=(seq // bs,),
        in_specs=[pl.BlockSpec((bs, d), lambda i: (i, 0))],
        out_specs=pl.BlockSpec((batch, bs, d), lambda i: (0, i, 0)),
        out_shape=jax.ShapeDtypeStruct((batch, seq, d), jnp.float32),
    )(table)
